# Initial kernel scaffold; baseline (speedup 1.0000x reference)
import jax
import jax.numpy as jnp
from jax.experimental import pallas as pl


def kernel(x, batched_edge_index, batched_edge_type, W1, root1, b1, W2, root2, b2):
    raise NotImplementedError("write your pallas kernel here")


# speedup vs baseline: 13.9931x; 13.9931x over previous
import functools

import jax
import jax.numpy as jnp
from jax import lax
from jax.experimental import pallas as pl
from jax.experimental.pallas import tpu as pltpu
from jax.experimental.pallas import tpu_sc as plsc

NN = 10000
RR = 8
DD = 128
EE = 320000
NB = 2
NC = 2
NS = 16
LL = 16
NW = NC * NS
SEG = NN * RR
EPT = EE // NW
CH = 80
NCHUNK = EPT // CH
GRP = CH // LL
RPT = NN // NS

_MESH = plsc.VectorSubcoreMesh(
    core_axis_name="c", subcore_axis_name="s", num_cores=NC, num_subcores=NS)
_SC_PARAMS = pltpu.CompilerParams(needs_layout_passes=False)


def _count_body(dst_hbm, et_hbm, out_hbm, dstb, etb, hist):
  cid = lax.axis_index("c")
  sid = lax.axis_index("s")
  wid = sid * NC + cid
  zeros = jnp.zeros((LL,), jnp.int32)
  ones = jnp.ones((LL,), jnp.int32)
  for b in range(NB):
    def zbody(i, _):
      hist[pl.ds(i * LL, LL)] = zeros
      return 0
    lax.fori_loop(0, SEG // LL, zbody, 0)
    base = b * EE + wid * EPT
    pltpu.sync_copy(dst_hbm.at[pl.ds(base, EPT)], dstb)
    pltpu.sync_copy(et_hbm.at[pl.ds(base, EPT)], etb)
    def cbody(i, _):
      sl = pl.ds(i * LL, LL)
      seg = dstb[sl] * RR + etb[sl]
      plsc.addupdate_scatter(hist, [seg], ones)
      return 0
    lax.fori_loop(0, EPT // LL, cbody, 0)
    pltpu.sync_copy(hist, out_hbm.at[pl.ds((b * NW + wid) * SEG, SEG)])


_count = pl.kernel(
    _count_body,
    out_type=jax.ShapeDtypeStruct((NB * NW * SEG,), jnp.int32),
    mesh=_MESH,
    scratch_types=[
        pltpu.VMEM((EPT,), jnp.int32),
        pltpu.VMEM((EPT,), jnp.int32),
        pltpu.VMEM((SEG,), jnp.int32),
    ],
    compiler_params=_SC_PARAMS,
)


SUP = 2000
NSUP = EPT // SUP
SUPCH = SUP // CH


def _scat_body(y_hbm, src_hbm, et_hbm, dst_hbm, inv_hbm, z_hbm,
               out_hbm, srcb, etb, dstb1, gb, segb, dstb, invb, rows, acc,
               gsem, isem):
  cid = lax.axis_index("c")
  sid = lax.axis_index("s")
  for b in range(NB):
    pltpu.sync_copy(z_hbm, acc.at[pl.ds(sid * RPT, RPT)])
    plsc.subcore_barrier()
    yb = y_hbm.at[b]
    invh = inv_hbm.at[pl.ds(b * SEG, SEG)]
    base = b * EE + cid * (EE // NC) + sid * EPT
    def sup_body(s, _):
      sbase = base + s * SUP
      pltpu.sync_copy(src_hbm.at[pl.ds(sbase, SUP)], srcb)
      pltpu.sync_copy(et_hbm.at[pl.ds(sbase, SUP)], etb)
      pltpu.sync_copy(dst_hbm.at[pl.ds(sbase, SUP)], dstb1)
      def gbody(i, _):
        sl = pl.ds(i * LL, LL)
        t = etb[sl]
        d = dstb1[sl]
        gb[sl] = srcb[sl] * RR + t
        segb[sl] = d * RR + t
        dstb[i // GRP, pl.ds((i % GRP) * LL, LL)] = d
        return 0
      lax.fori_loop(0, SUP // LL, gbody, 0)
      def cbody(i, _):
        pltpu.async_copy(yb.at[gb.at[pl.ds(i * CH, CH)]], rows, gsem).wait()
        pltpu.async_copy(invh.at[segb.at[pl.ds(i * CH, CH)]], invb, isem).wait()
        def sc_body(k, _):
          for j in range(LL):
            e = k * LL + j
            scal = plsc.load_gather(invb, [jnp.full((LL,), e, jnp.int32)])
            for q in range(DD // LL):
              sl = pl.ds(q * LL, LL)
              rows[e, sl] = rows[e, sl] * scal
          return 0
        lax.fori_loop(0, GRP, sc_body, 0)
        pltpu.sync_copy(rows, acc.at[dstb.at[i]], add=True)
        return 0
      lax.fori_loop(0, SUPCH, cbody, 0)
      return 0
    lax.fori_loop(0, NSUP, sup_body, 0)
    plsc.subcore_barrier()
    pltpu.sync_copy(acc.at[pl.ds(sid * RPT, RPT)], out_hbm.at[b, cid, sid])


_scat = pl.kernel(
    _scat_body,
    out_type=jax.ShapeDtypeStruct((NB, NC, NS, RPT, DD), jnp.float32),
    mesh=_MESH,
    scratch_types=[
        pltpu.VMEM((SUP,), jnp.int32),
        pltpu.VMEM((SUP,), jnp.int32),
        pltpu.VMEM((SUP,), jnp.int32),
        pltpu.VMEM((SUP,), jnp.int32),
        pltpu.VMEM((SUP,), jnp.int32),
        pltpu.VMEM((SUPCH, CH), jnp.int32),
        pltpu.VMEM((CH,), jnp.float32),
        pltpu.VMEM((CH, DD), jnp.float32),
        pltpu.VMEM_SHARED((NN, DD), jnp.float32),
        pltpu.SemaphoreType.DMA,
        pltpu.SemaphoreType.DMA,
    ],
    compiler_params=_SC_PARAMS,
)


def _inv_body(cnt_ref, inv_ref):
  s = jnp.sum(cnt_ref[0], axis=0)
  inv_ref[...] = (1.0 / jnp.maximum(s, 1).astype(jnp.float32))[None]


def _inv(cnt):
  cnt = cnt.reshape(NB, NW, SEG // DD, DD)
  out = pl.pallas_call(
      _inv_body,
      out_shape=jax.ShapeDtypeStruct((NB, SEG // DD, DD), jnp.float32),
      grid=(NB,),
      in_specs=[pl.BlockSpec((1, NW, SEG // DD, DD), lambda b: (b, 0, 0, 0))],
      out_specs=pl.BlockSpec((1, SEG // DD, DD), lambda b: (b, 0, 0)),
  )(cnt)
  return out.reshape(NB, SEG)


_BM = 2000


def _mm_body(x_ref, w_ref, r_ref, bias_ref, y_ref, xr_ref):
  xb = x_ref[0]
  y_ref[...] = jnp.dot(xb, w_ref[...], preferred_element_type=jnp.float32)[None]
  xr_ref[...] = (jnp.dot(xb, r_ref[...], preferred_element_type=jnp.float32)
                 + bias_ref[...])[None]


def _mm(x, wc, root, bias):
  return pl.pallas_call(
      _mm_body,
      out_shape=(
          jax.ShapeDtypeStruct((NB, NN, RR * DD), jnp.float32),
          jax.ShapeDtypeStruct((NB, NN, DD), jnp.float32),
      ),
      grid=(NB, NN // _BM),
      in_specs=[
          pl.BlockSpec((1, _BM, DD), lambda b, i: (b, i, 0)),
          pl.BlockSpec((DD, RR * DD), lambda b, i: (0, 0)),
          pl.BlockSpec((DD, DD), lambda b, i: (0, 0)),
          pl.BlockSpec((1, DD), lambda b, i: (0, 0)),
      ],
      out_specs=(
          pl.BlockSpec((1, _BM, RR * DD), lambda b, i: (b, i, 0)),
          pl.BlockSpec((1, _BM, DD), lambda b, i: (b, i, 0)),
      ),
  )(x, wc, root, bias)


def _fuse_body(part_ref, xr_ref, o_ref, *, relu):
  s = part_ref[0, 0] + part_ref[0, 1] + xr_ref[0]
  if relu:
    s = jnp.maximum(s, 0.0)
  o_ref[...] = s[None]


def _fuse(parts, xr, relu):
  return pl.pallas_call(
      functools.partial(_fuse_body, relu=relu),
      out_shape=jax.ShapeDtypeStruct((NB, NN, DD), jnp.float32),
      grid=(NB, NN // _BM),
      in_specs=[
          pl.BlockSpec((1, NC, _BM, DD), lambda b, i: (b, 0, i, 0)),
          pl.BlockSpec((1, _BM, DD), lambda b, i: (b, i, 0)),
      ],
      out_specs=pl.BlockSpec((1, _BM, DD), lambda b, i: (b, i, 0)),
  )(parts, xr)


def kernel(x, batched_edge_index, batched_edge_type, W1, root1, b1,
           W2, root2, b2):
  src = batched_edge_index[:, 0, :].astype(jnp.int32).reshape(NB * EE)
  dst = batched_edge_index[:, 1, :].astype(jnp.int32).reshape(NB * EE)
  et = batched_edge_type.astype(jnp.int32).reshape(NB * EE)
  zrows = jnp.zeros((RPT, DD), jnp.float32)

  cnt = _count(dst, et)
  inv = _inv(cnt).reshape(NB * SEG)

  wc1 = W1.transpose(1, 0, 2).reshape(DD, RR * DD)
  wc2 = W2.transpose(1, 0, 2).reshape(DD, RR * DD)

  y1, xr1 = _mm(x, wc1, root1, b1.reshape(1, DD))
  parts1 = _scat(y1.reshape(NB, SEG, DD), src, et, dst, inv, zrows)
  h = _fuse(parts1.reshape(NB, NC, NN, DD), xr1, relu=True)

  y2, xr2 = _mm(h, wc2, root2, b2.reshape(1, DD))
  parts2 = _scat(y2.reshape(NB, SEG, DD), src, et, dst, inv, zrows)
  return _fuse(parts2.reshape(NB, NC, NN, DD), xr2, relu=False)
